# baseline (speedup 1.0000x reference)
import functools

import jax
import jax.numpy as jnp
from jax.experimental import pallas as pl
from jax.experimental.pallas import tpu as pltpu

_C_OUT = 10240
_SPLIT = 4
_C_IN = _C_OUT // _SPLIT


def _oim_body(f_ref, t_ref, l1_ref, l2_ref, l3_ref, l4_ref,
              logits_ref, loss_ref, m_ref, s_ref, tl_ref,
              *, nout, nclasses):
    i = pl.program_id(0)
    j = pl.program_id(1)

    @pl.when(jnp.logical_and(i == 0, j == 0))
    def _init():
        m_ref[...] = jnp.full_like(m_ref, -jnp.inf)
        s_ref[...] = jnp.zeros_like(s_ref)
        tl_ref[...] = jnp.zeros_like(tl_ref)

    f = f_ref[...]
    dn = (((1,), (1,)), ((), ()))
    acc = jax.lax.dot_general(f[:, 0, :], l1_ref[...], dn,
                              preferred_element_type=jnp.float32)
    acc += jax.lax.dot_general(f[:, 1, :], l2_ref[...], dn,
                               preferred_element_type=jnp.float32)
    acc += jax.lax.dot_general(f[:, 2, :], l3_ref[...], dn,
                               preferred_element_type=jnp.float32)
    acc += jax.lax.dot_general(f[:, 3, :], l4_ref[...], dn,
                               preferred_element_type=jnp.float32)
    logits_ref[:, pl.ds(j * _C_IN, _C_IN)] = acc

    col = (jax.lax.broadcasted_iota(jnp.int32, acc.shape, 1)
           + (i * _SPLIT + j) * _C_IN)
    valid = col < nclasses
    masked = jnp.where(valid, acc, -jnp.inf)
    bmax = jnp.max(masked, axis=1, keepdims=True)
    m_old = m_ref[...]
    m_new = jnp.maximum(m_old, bmax)
    p = jnp.where(valid, jnp.exp(acc - m_new), 0.0)
    s_ref[...] = s_ref[...] * jnp.exp(m_old - m_new) + jnp.sum(
        p, axis=1, keepdims=True)
    m_ref[...] = m_new

    t = t_ref[...]
    tl_ref[...] += jnp.sum(jnp.where(col == t, acc, 0.0), axis=1,
                           keepdims=True)

    @pl.when(jnp.logical_and(i == nout - 1, j == _SPLIT - 1))
    def _fin():
        lse = m_ref[...] + jnp.log(s_ref[...])
        loss_ref[...] = jnp.mean(lse - tl_ref[...]).reshape(1, 1)


def kernel(features, scores, targets, flags, lut_b1, lut_b2, lut_b3,
           lut_b4):
    batch, _, nfeat = features.shape
    nclasses = lut_b1.shape[0]
    nout = pl.cdiv(nclasses, _C_OUT)
    t2 = targets.astype(jnp.int32).reshape(batch, 1)

    body = functools.partial(_oim_body, nout=nout, nclasses=nclasses)
    logits, loss = pl.pallas_call(
        body,
        grid=(nout, _SPLIT),
        in_specs=[
            pl.BlockSpec((batch, 4, nfeat), lambda i, j: (0, 0, 0)),
            pl.BlockSpec((batch, 1), lambda i, j: (0, 0)),
            pl.BlockSpec((_C_IN, nfeat), lambda i, j: (i * _SPLIT + j, 0)),
            pl.BlockSpec((_C_IN, nfeat), lambda i, j: (i * _SPLIT + j, 0)),
            pl.BlockSpec((_C_IN, nfeat), lambda i, j: (i * _SPLIT + j, 0)),
            pl.BlockSpec((_C_IN, nfeat), lambda i, j: (i * _SPLIT + j, 0)),
        ],
        out_specs=[
            pl.BlockSpec((batch, _C_OUT), lambda i, j: (0, i)),
            pl.BlockSpec((1, 1), lambda i, j: (0, 0)),
        ],
        out_shape=[
            jax.ShapeDtypeStruct((batch, nclasses), jnp.float32),
            jax.ShapeDtypeStruct((1, 1), jnp.float32),
        ],
        scratch_shapes=[
            pltpu.VMEM((batch, 1), jnp.float32),
            pltpu.VMEM((batch, 1), jnp.float32),
            pltpu.VMEM((batch, 1), jnp.float32),
        ],
        compiler_params=pltpu.CompilerParams(
            dimension_semantics=("arbitrary", "arbitrary")),
    )(features, t2, lut_b1, lut_b2, lut_b3, lut_b4)
    return (loss[0, 0], logits)


# speedup vs baseline: 1.2227x; 1.2227x over previous
import functools

import jax
import jax.numpy as jnp
from jax.experimental import pallas as pl
from jax.experimental.pallas import tpu as pltpu

_NUM_CLASSES = 100000
_C_BLK = 10240


def _oim_body(f_ref, t_ref, l1_ref, l2_ref, l3_ref, l4_ref,
              logits_ref, loss_ref, m_ref, s_ref, tl_ref,
              *, nblk, nclasses, cblk):
    i = pl.program_id(0)

    @pl.when(i == 0)
    def _init():
        m_ref[...] = jnp.full_like(m_ref, -jnp.inf)
        s_ref[...] = jnp.zeros_like(s_ref)
        tl_ref[...] = jnp.zeros_like(tl_ref)

    f = f_ref[...].astype(jnp.bfloat16)
    dn = (((1,), (1,)), ((), ()))
    acc = jax.lax.dot_general(f[:, 0, :], l1_ref[...].astype(jnp.bfloat16),
                              dn, preferred_element_type=jnp.float32)
    acc += jax.lax.dot_general(f[:, 1, :], l2_ref[...].astype(jnp.bfloat16),
                               dn, preferred_element_type=jnp.float32)
    acc += jax.lax.dot_general(f[:, 2, :], l3_ref[...].astype(jnp.bfloat16),
                               dn, preferred_element_type=jnp.float32)
    acc += jax.lax.dot_general(f[:, 3, :], l4_ref[...].astype(jnp.bfloat16),
                               dn, preferred_element_type=jnp.float32)
    logits_ref[...] = acc

    col = jax.lax.broadcasted_iota(jnp.int32, acc.shape, 1) + i * cblk
    valid = col < nclasses
    masked = jnp.where(valid, acc, -jnp.inf)
    bmax = jnp.max(masked, axis=1, keepdims=True)
    m_old = m_ref[...]
    m_new = jnp.maximum(m_old, bmax)
    p = jnp.where(valid, jnp.exp(acc - m_new), 0.0)
    s_ref[...] = s_ref[...] * jnp.exp(m_old - m_new) + jnp.sum(
        p, axis=1, keepdims=True)
    m_ref[...] = m_new

    t = t_ref[...]
    tl_ref[...] += jnp.sum(jnp.where(col == t, acc, 0.0), axis=1,
                           keepdims=True)

    @pl.when(i == nblk - 1)
    def _fin():
        lse = m_ref[...] + jnp.log(s_ref[...])
        loss_ref[...] = jnp.mean(lse - tl_ref[...]).reshape(1, 1)


def kernel(features, scores, targets, flags, lut_b1, lut_b2, lut_b3,
           lut_b4):
    batch, _, nfeat = features.shape
    nclasses = lut_b1.shape[0]
    nblk = pl.cdiv(nclasses, _C_BLK)
    t2 = targets.astype(jnp.int32).reshape(batch, 1)

    body = functools.partial(_oim_body, nblk=nblk, nclasses=nclasses,
                             cblk=_C_BLK)
    logits, loss = pl.pallas_call(
        body,
        grid=(nblk,),
        in_specs=[
            pl.BlockSpec((batch, 4, nfeat), lambda i: (0, 0, 0)),
            pl.BlockSpec((batch, 1), lambda i: (0, 0)),
            pl.BlockSpec((_C_BLK, nfeat), lambda i: (i, 0)),
            pl.BlockSpec((_C_BLK, nfeat), lambda i: (i, 0)),
            pl.BlockSpec((_C_BLK, nfeat), lambda i: (i, 0)),
            pl.BlockSpec((_C_BLK, nfeat), lambda i: (i, 0)),
        ],
        out_specs=[
            pl.BlockSpec((batch, _C_BLK), lambda i: (0, i)),
            pl.BlockSpec((1, 1), lambda i: (0, 0)),
        ],
        out_shape=[
            jax.ShapeDtypeStruct((batch, nclasses), jnp.float32),
            jax.ShapeDtypeStruct((1, 1), jnp.float32),
        ],
        scratch_shapes=[
            pltpu.VMEM((batch, 1), jnp.float32),
            pltpu.VMEM((batch, 1), jnp.float32),
            pltpu.VMEM((batch, 1), jnp.float32),
        ],
        compiler_params=pltpu.CompilerParams(
            dimension_semantics=("arbitrary",)),
    )(features, t2, lut_b1, lut_b2, lut_b3, lut_b4)
    return (loss[0, 0], logits)


# speedup vs baseline: 1.2281x; 1.0044x over previous
import jax
import jax.numpy as jnp
from jax.experimental import pallas as pl
from jax.experimental.pallas import tpu as pltpu

_CH = 4096
_K = 3
_NCH = 24
_TAIL = 1696
_TSTART = _NCH * _CH


def _oim_body(f_ref, t_ref, l1_ref, l2_ref, l3_ref, l4_ref,
              logits_ref, loss_ref,
              b1_ref, b2_ref, b3_ref, b4_ref, stage_ref,
              tb1_ref, tb2_ref, tb3_ref, tb4_ref, tstage_ref,
              sem_in, sem_out, sem_tin, sem_tout):
    lut_refs = (l1_ref, l2_ref, l3_ref, l4_ref)
    buf_refs = (b1_ref, b2_ref, b3_ref, b4_ref)
    tbuf_refs = (tb1_ref, tb2_ref, tb3_ref, tb4_ref)

    def in_copy(b, c):
        return pltpu.make_async_copy(
            lut_refs[b].at[pl.ds(c * _CH, _CH), :],
            buf_refs[b].at[c % _K],
            sem_in.at[b, c % _K])

    def out_copy(c):
        return pltpu.make_async_copy(
            stage_ref.at[c % _K],
            logits_ref.at[:, pl.ds(c * _CH, _CH)],
            sem_out.at[c % _K])

    def tin_copy(b):
        return pltpu.make_async_copy(
            lut_refs[b].at[pl.ds(_TSTART, _TAIL), :],
            tbuf_refs[b],
            sem_tin.at[b])

    def tout_copy():
        return pltpu.make_async_copy(
            tstage_ref,
            logits_ref.at[:, pl.ds(_TSTART, _TAIL)],
            sem_tout)

    for c in range(_K):
        for b in range(4):
            in_copy(b, c).start()
    for b in range(4):
        tin_copy(b).start()

    f = f_ref[...].astype(jnp.float32)
    t = t_ref[...]
    batch = f.shape[0]
    dn = (((1,), (1,)), ((), ()))

    m = jnp.full((batch, 1), -jnp.inf, dtype=jnp.float32)
    s = jnp.zeros((batch, 1), dtype=jnp.float32)
    tl = jnp.zeros((batch, 1), dtype=jnp.float32)

    def fold(m, s, tl, acc, base):
        col = jax.lax.broadcasted_iota(jnp.int32, acc.shape, 1) + base
        bmax = jnp.max(acc, axis=1, keepdims=True)
        m_new = jnp.maximum(m, bmax)
        p = jnp.exp(acc - m_new)
        s = s * jnp.exp(m - m_new) + jnp.sum(p, axis=1, keepdims=True)
        tl = tl + jnp.sum(jnp.where(col == t, acc, 0.0), axis=1,
                          keepdims=True)
        return m_new, s, tl

    for c in range(_NCH):
        slot = c % _K
        for b in range(4):
            in_copy(b, c).wait()
        acc = jax.lax.dot_general(f[:, 0, :], b1_ref[slot], dn,
                                  preferred_element_type=jnp.float32)
        acc += jax.lax.dot_general(f[:, 1, :], b2_ref[slot], dn,
                                   preferred_element_type=jnp.float32)
        acc += jax.lax.dot_general(f[:, 2, :], b3_ref[slot], dn,
                                   preferred_element_type=jnp.float32)
        acc += jax.lax.dot_general(f[:, 3, :], b4_ref[slot], dn,
                                   preferred_element_type=jnp.float32)
        if c >= _K:
            out_copy(c - _K).wait()
        stage_ref[slot] = acc
        out_copy(c).start()
        if c + _K < _NCH:
            for b in range(4):
                in_copy(b, c + _K).start()
        m, s, tl = fold(m, s, tl, acc, c * _CH)

    for b in range(4):
        tin_copy(b).wait()
    acc = jax.lax.dot_general(f[:, 0, :], tb1_ref[...], dn,
                              preferred_element_type=jnp.float32)
    acc += jax.lax.dot_general(f[:, 1, :], tb2_ref[...], dn,
                               preferred_element_type=jnp.float32)
    acc += jax.lax.dot_general(f[:, 2, :], tb3_ref[...], dn,
                               preferred_element_type=jnp.float32)
    acc += jax.lax.dot_general(f[:, 3, :], tb4_ref[...], dn,
                               preferred_element_type=jnp.float32)
    tstage_ref[...] = acc
    tout_copy().start()
    m, s, tl = fold(m, s, tl, acc, _TSTART)

    for c in range(_NCH - _K, _NCH):
        out_copy(c).wait()
    tout_copy().wait()

    loss_ref[...] = jnp.mean(m + jnp.log(s) - tl).reshape(1, 1)


def kernel(features, scores, targets, flags, lut_b1, lut_b2, lut_b3,
           lut_b4):
    batch, _, nfeat = features.shape
    nclasses = lut_b1.shape[0]
    t2 = targets.astype(jnp.int32).reshape(batch, 1)

    lut_spec = pl.BlockSpec(memory_space=pl.ANY)
    logits, loss = pl.pallas_call(
        _oim_body,
        in_specs=[
            pl.BlockSpec(memory_space=pltpu.MemorySpace.VMEM),
            pl.BlockSpec(memory_space=pltpu.MemorySpace.VMEM),
            lut_spec, lut_spec, lut_spec, lut_spec,
        ],
        out_specs=[
            pl.BlockSpec(memory_space=pl.ANY),
            pl.BlockSpec(memory_space=pltpu.MemorySpace.VMEM),
        ],
        out_shape=[
            jax.ShapeDtypeStruct((batch, nclasses), jnp.float32),
            jax.ShapeDtypeStruct((1, 1), jnp.float32),
        ],
        scratch_shapes=[
            pltpu.VMEM((_K, _CH, nfeat), jnp.float32),
            pltpu.VMEM((_K, _CH, nfeat), jnp.float32),
            pltpu.VMEM((_K, _CH, nfeat), jnp.float32),
            pltpu.VMEM((_K, _CH, nfeat), jnp.float32),
            pltpu.VMEM((_K, batch, _CH), jnp.float32),
            pltpu.VMEM((_TAIL, nfeat), jnp.float32),
            pltpu.VMEM((_TAIL, nfeat), jnp.float32),
            pltpu.VMEM((_TAIL, nfeat), jnp.float32),
            pltpu.VMEM((_TAIL, nfeat), jnp.float32),
            pltpu.VMEM((batch, _TAIL), jnp.float32),
            pltpu.SemaphoreType.DMA((4, _K)),
            pltpu.SemaphoreType.DMA((_K,)),
            pltpu.SemaphoreType.DMA((4,)),
            pltpu.SemaphoreType.DMA,
        ],
    )(features, t2, lut_b1, lut_b2, lut_b3, lut_b4)
    return (loss[0, 0], logits)
